# Initial kernel scaffold; baseline (speedup 1.0000x reference)
import jax
import jax.numpy as jnp
from jax.experimental import pallas as pl


def kernel(ap_x, fl_x, ap_h, params, edge0, edge1, edge2, edge3, edge4, edge5):
    raise NotImplementedError("write your pallas kernel here")


# speedup vs baseline: 3.4588x; 3.4588x over previous
import functools

import jax
import jax.numpy as jnp
import numpy as np
from jax import lax
from jax.experimental import pallas as pl
from jax.experimental.pallas import tpu as pltpu
from jax.experimental.pallas import tpu_sc as plsc

N_AP = 10000
N_FL = 50000
HID = 256
HEADS = 8
DH = 32
LAYERS = 2
E = 50000
EP = 51200
NE = 6
ET = NE * EP
CH = 80
NC = 2
NS = 16
NW = NC * NS

_LOFF = [0, 10000, 20000, 0, 0, 10000]
_LDUM = [30000, 30000, 30000, 50000, 20000, 20000]
_R8 = (3840, 6400, 2560)
_PH8 = ((_R8[0], 0, 3 * EP),
        (_R8[1], 3 * EP, EP),
        (_R8[2], 4 * EP, 2 * EP))
_PH8BASE = (0, 3840, 10240)
R8TOT = 12800
_G8BASE = [0, 0, 0, 3840, 10240, 10240]
AP_ROWS = 10240
FL_ROWS = 50176

_f32 = jnp.float32
_i32 = jnp.int32

_SEG16 = np.zeros((HID, 16), np.float32)
for _h in range(HEADS):
    _SEG16[_h * DH:(_h + 1) * DH, _h] = 1.0
_EXP16 = np.zeros((16, HID), np.float32)
for _h in range(HEADS):
    _EXP16[_h, _h * DH:(_h + 1) * DH] = 1.0
_SH16 = np.zeros((16, 16), np.float32)
for _h in range(8):
    _SH16[8 + _h, _h] = 1.0
_S128 = np.zeros((128, 16), np.float32)
for _i in range(128):
    _S128[_i, _i % 16] = 1.0

_DOT = functools.partial(jnp.dot, preferred_element_type=jnp.float32,
                         precision=lax.Precision.HIGHEST)


def _projln_body(x_ref, w_ref, b_ref, g_ref, be_ref, o_ref):
    h = _DOT(x_ref[...], w_ref[...]) + b_ref[...]
    m = jnp.mean(h, axis=-1, keepdims=True)
    v = jnp.mean((h - m) ** 2, axis=-1, keepdims=True)
    o_ref[...] = (h - m) / jnp.sqrt(v + 1e-5) * g_ref[...] + be_ref[...]


def _projln(x, w, b, g, be, bm):
    m, k = x.shape
    return pl.pallas_call(
        _projln_body,
        grid=(m // bm,),
        in_specs=[
            pl.BlockSpec((bm, k), lambda i: (i, 0)),
            pl.BlockSpec((k, HID), lambda i: (0, 0)),
            pl.BlockSpec((1, HID), lambda i: (0, 0)),
            pl.BlockSpec((1, HID), lambda i: (0, 0)),
            pl.BlockSpec((1, HID), lambda i: (0, 0)),
        ],
        out_specs=pl.BlockSpec((bm, HID), lambda i: (i, 0)),
        out_shape=jax.ShapeDtypeStruct((m, HID), _f32),
    )(x, w, b[None, :], g[None, :], be[None, :])


def _bigmat_body(x_ref, w_ref, b_ref, *o_refs):
    x = x_ref[...]
    for j in range(7):
        o_refs[j][...] = (_DOT(x, w_ref[:, j * HID:(j + 1) * HID])
                          + b_ref[:, j * HID:(j + 1) * HID])


def _bigmat(x, w, b, bm):
    m = x.shape[0]
    return pl.pallas_call(
        _bigmat_body,
        grid=(m // bm,),
        in_specs=[
            pl.BlockSpec((bm, HID), lambda i: (i, 0)),
            pl.BlockSpec((HID, 7 * HID), lambda i: (0, 0)),
            pl.BlockSpec((1, 7 * HID), lambda i: (0, 0)),
        ],
        out_specs=[pl.BlockSpec((bm, HID), lambda i: (i, 0))] * 7,
        out_shape=[jax.ShapeDtypeStruct((m, HID), _f32)] * 7,
    )(x, w, b[None, :])


def _place128(x16, idx2, tile):
    x128 = _DOT(x16, tile)
    blk = lax.broadcasted_iota(_i32, x128.shape, 1) // 16
    return jnp.where(blk == (idx2 & 7), x128, 0.0)


def _alpha_body(qi_ref, kj_ref, i_ref, seg_ref, tile_ref, o_ref, o128_ref):
    p = _DOT(qi_ref[...] * kj_ref[...], seg_ref[...])
    lane = lax.broadcasted_iota(_i32, p.shape, 1)
    a16 = jnp.where(lane >= 8, 1.0, p)
    o_ref[...] = a16
    o128_ref[...] = _place128(a16, i_ref[...], tile_ref[...])


def _alpha16(qi, kj, idxl2, bm):
    return pl.pallas_call(
        _alpha_body,
        grid=(ET // bm,),
        in_specs=[
            pl.BlockSpec((bm, HID), lambda i: (i, 0)),
            pl.BlockSpec((bm, HID), lambda i: (i, 0)),
            pl.BlockSpec((bm, 1), lambda i: (i, 0)),
            pl.BlockSpec((HID, 16), lambda i: (0, 0)),
            pl.BlockSpec((16, 128), lambda i: (0, 0)),
        ],
        out_specs=[pl.BlockSpec((bm, 16), lambda i: (i, 0)),
                   pl.BlockSpec((bm, 128), lambda i: (i, 0))],
        out_shape=[jax.ShapeDtypeStruct((ET, 16), _f32),
                   jax.ShapeDtypeStruct((ET, 128), _f32)],
    )(qi, kj, idxl2, jnp.asarray(_SEG16), jnp.asarray(_S128.T.copy()))


def _sel16(g, idx2, fold):
    blk = lax.broadcasted_iota(_i32, g.shape, 1) // 16
    mask = jnp.where(blk == (idx2 & 7), 1.0, 0.0)
    return _DOT(g * mask, fold)


def _e_body(a_ref, g_ref, i_ref, f_ref, sh_ref, tile_ref, o_ref, o128_ref):
    s = _sel16(g_ref[...], i_ref[...], f_ref[...])
    csh = _DOT(s, sh_ref[...])
    mean = s / jnp.maximum(csh, 1.0)
    d = a_ref[...] - mean
    lane = lax.broadcasted_iota(_i32, d.shape, 1)
    e16 = jnp.where(lane < 8, jnp.exp(jnp.minimum(d, 60.0)), 0.0)
    o_ref[...] = e16
    o128_ref[...] = _place128(e16, i_ref[...], tile_ref[...])


def _e16(a16, g, idxl2, bm):
    return pl.pallas_call(
        _e_body,
        grid=(ET // bm,),
        in_specs=[
            pl.BlockSpec((bm, 16), lambda i: (i, 0)),
            pl.BlockSpec((bm, 128), lambda i: (i, 0)),
            pl.BlockSpec((bm, 1), lambda i: (i, 0)),
            pl.BlockSpec((128, 16), lambda i: (0, 0)),
            pl.BlockSpec((16, 16), lambda i: (0, 0)),
            pl.BlockSpec((16, 128), lambda i: (0, 0)),
        ],
        out_specs=[pl.BlockSpec((bm, 16), lambda i: (i, 0)),
                   pl.BlockSpec((bm, 128), lambda i: (i, 0))],
        out_shape=[jax.ShapeDtypeStruct((ET, 16), _f32),
                   jax.ShapeDtypeStruct((ET, 128), _f32)],
    )(a16, g, idxl2, jnp.asarray(_S128), jnp.asarray(_SH16),
      jnp.asarray(_S128.T.copy()))


def _msg_body(vj_ref, e_ref, g_ref, i_ref, f_ref, ex_ref, o0_ref, o1_ref):
    esum = _sel16(g_ref[...], i_ref[...], f_ref[...])
    w = e_ref[...] / (esum + 1e-16)
    o = vj_ref[...] * _DOT(w, ex_ref[...])
    o0_ref[...] = o[:, :128]
    o1_ref[...] = o[:, 128:]


def _msg(vj, e16, g, idxl2, bm):
    return pl.pallas_call(
        _msg_body,
        grid=(ET // bm,),
        in_specs=[
            pl.BlockSpec((bm, HID), lambda i: (i, 0)),
            pl.BlockSpec((bm, 16), lambda i: (i, 0)),
            pl.BlockSpec((bm, 128), lambda i: (i, 0)),
            pl.BlockSpec((bm, 1), lambda i: (i, 0)),
            pl.BlockSpec((128, 16), lambda i: (0, 0)),
            pl.BlockSpec((16, HID), lambda i: (0, 0)),
        ],
        out_specs=[pl.BlockSpec((bm, 128), lambda i: (i, 0))] * 2,
        out_shape=[jax.ShapeDtypeStruct((ET, 128), _f32)] * 2,
    )(vj, e16, g, idxl2, jnp.asarray(_S128), jnp.asarray(_EXP16))


def _nodeup_body(agg_ref, x_ref, w_ref, b_ref, a_ref, g_ref, bb_ref, o_ref):
    a4 = agg_ref[...]
    agg = jnp.concatenate(
        [a4[0, 0] + a4[1, 0], a4[0, 1] + a4[1, 1]], axis=-1)
    o1 = _DOT(jax.nn.gelu(agg), w_ref[...]) + b_ref[...]
    a = a_ref[0, 0]
    x = x_ref[...]
    h = a * o1 + (1.0 - a) * x + x
    m = jnp.mean(h, axis=-1, keepdims=True)
    v = jnp.mean((h - m) ** 2, axis=-1, keepdims=True)
    o_ref[...] = (h - m) / jnp.sqrt(v + 1e-5) * g_ref[...] + bb_ref[...]


def _node_update(agg4, x, w, b, a_sig, g, bb, bm):
    m = x.shape[0]
    return pl.pallas_call(
        _nodeup_body,
        grid=(m // bm,),
        in_specs=[
            pl.BlockSpec((2, 2, bm, 128), lambda i: (0, 0, i, 0)),
            pl.BlockSpec((bm, HID), lambda i: (i, 0)),
            pl.BlockSpec((HID, HID), lambda i: (0, 0)),
            pl.BlockSpec((1, HID), lambda i: (0, 0)),
            pl.BlockSpec((1, 128), lambda i: (0, 0)),
            pl.BlockSpec((1, HID), lambda i: (0, 0)),
            pl.BlockSpec((1, HID), lambda i: (0, 0)),
        ],
        out_specs=pl.BlockSpec((bm, HID), lambda i: (i, 0)),
        out_shape=jax.ShapeDtypeStruct((m, HID), _f32),
    )(agg4, x, w, b[None, :], a_sig, g[None, :], bb[None, :])


def _gru_body(x_ref, h_ref, wih_ref, whh_ref, bih_ref, bhh_ref,
              w1_ref, b1_ref, w2_ref, b2_ref, hn_ref, p_ref):
    gi = _DOT(x_ref[...], wih_ref[...]) + bih_ref[...]
    gh = _DOT(h_ref[...], whh_ref[...]) + bhh_ref[...]
    r = jax.nn.sigmoid(gi[:, :HID] + gh[:, :HID])
    z = jax.nn.sigmoid(gi[:, HID:2 * HID] + gh[:, HID:2 * HID])
    n = jnp.tanh(gi[:, 2 * HID:] + r * gh[:, 2 * HID:])
    hn = (1.0 - z) * n + z * h_ref[...]
    hn_ref[...] = hn
    p_ref[...] = _DOT(jax.nn.relu(_DOT(hn, w1_ref[...]) + b1_ref[...]),
                      w2_ref[...]) + b2_ref[...]


def _gru_head(x, h, wih, whh, bih, bhh, w1, b1, w2p, b2p, bm):
    m = x.shape[0]
    return pl.pallas_call(
        _gru_body,
        grid=(m // bm,),
        in_specs=[
            pl.BlockSpec((bm, HID), lambda i: (i, 0)),
            pl.BlockSpec((bm, HID), lambda i: (i, 0)),
            pl.BlockSpec((HID, 3 * HID), lambda i: (0, 0)),
            pl.BlockSpec((HID, 3 * HID), lambda i: (0, 0)),
            pl.BlockSpec((1, 3 * HID), lambda i: (0, 0)),
            pl.BlockSpec((1, 3 * HID), lambda i: (0, 0)),
            pl.BlockSpec((HID, 128), lambda i: (0, 0)),
            pl.BlockSpec((1, 128), lambda i: (0, 0)),
            pl.BlockSpec((128, 128), lambda i: (0, 0)),
            pl.BlockSpec((1, 128), lambda i: (0, 0)),
        ],
        out_specs=[pl.BlockSpec((bm, HID), lambda i: (i, 0)),
                   pl.BlockSpec((bm, 128), lambda i: (i, 0))],
        out_shape=[jax.ShapeDtypeStruct((m, HID), _f32),
                   jax.ShapeDtypeStruct((m, 128), _f32)],
    )(x, h, wih, whh, bih[None, :], bhh[None, :], w1, b1[None, :], w2p, b2p)


def _flhead_body(x_ref, gw_ref, gb_ref, w1_ref, b1_ref, w2_ref,
                 cw1_ref, cb1_ref, cw2_ref, p_ref, l_ref):
    x = x_ref[...]
    g = x * jax.nn.sigmoid(_DOT(x, gw_ref[...]) + gb_ref[...])
    p_ref[...] = _DOT(jax.nn.relu(_DOT(g, w1_ref[...]) + b1_ref[...]),
                      w2_ref[...])
    l_ref[...] = _DOT(jax.nn.relu(_DOT(g, cw1_ref[...]) + cb1_ref[...]),
                      cw2_ref[...])


def _fl_head(x, gw, gb, w1, b1, w2p, cw1p, cb1p, cw2p, bm):
    m = x.shape[0]
    return pl.pallas_call(
        _flhead_body,
        grid=(m // bm,),
        in_specs=[
            pl.BlockSpec((bm, HID), lambda i: (i, 0)),
            pl.BlockSpec((HID, HID), lambda i: (0, 0)),
            pl.BlockSpec((1, HID), lambda i: (0, 0)),
            pl.BlockSpec((HID, 128), lambda i: (0, 0)),
            pl.BlockSpec((1, 128), lambda i: (0, 0)),
            pl.BlockSpec((128, 128), lambda i: (0, 0)),
            pl.BlockSpec((HID, 128), lambda i: (0, 0)),
            pl.BlockSpec((1, 128), lambda i: (0, 0)),
            pl.BlockSpec((128, 128), lambda i: (0, 0)),
        ],
        out_specs=[pl.BlockSpec((bm, 128), lambda i: (i, 0)),
                   pl.BlockSpec((bm, 128), lambda i: (i, 0))],
        out_shape=[jax.ShapeDtypeStruct((m, 128), _f32),
                   jax.ShapeDtypeStruct((m, 128), _f32)],
    )(x, gw, gb[None, :], w1, b1[None, :], w2p, cw1p, cb1p, cw2p)


_MESH = plsc.VectorSubcoreMesh(core_axis_name="c", subcore_axis_name="s")


def _wid():
    return lax.axis_index("s") * NC + lax.axis_index("c")


def _g1_body(qap, qfl, kr0, kr1, kr2, kr3, kr4, kr5,
             vr0, vr1, vr2, vr3, vr4, vr5, dstg, srcg,
             qi, kj, vj, idx_v, row_v, sem):
    w = _wid()
    krs = [kr0, kr1, kr2, kr3, kr4, kr5]
    vrs = [vr0, vr1, vr2, vr3, vr4, vr5]
    jobs = [(qap, dstg, qi, 0, 3 * EP),
            (qfl, dstg, qi, 3 * EP, EP),
            (qap, dstg, qi, 4 * EP, 2 * EP)]
    jobs += [(krs[e], srcg, kj, e * EP, EP) for e in range(NE)]
    jobs += [(vrs[e], srcg, vj, e * EP, EP) for e in range(NE)]
    for table, idxarr, out, lo, ln in jobs:
        per_w = ln // NW
        base0 = lo + w * per_w

        @pl.loop(0, per_w // CH)
        def _chunk(i):
            base = base0 + i * CH
            pltpu.sync_copy(idxarr.at[pl.ds(base, CH)], idx_v)
            pltpu.async_copy(table.at[idx_v], row_v, sem).wait()
            pltpu.sync_copy(row_v, out.at[pl.ds(base, CH)])


def _g1(qap, qfl, krs, vrs, dstg, srcg):
    f = pl.kernel(
        _g1_body,
        out_type=[jax.ShapeDtypeStruct((ET, HID), _f32)] * 3,
        mesh=_MESH,
        scratch_types=[pltpu.VMEM((CH,), _i32),
                       pltpu.VMEM((CH, HID), _f32),
                       pltpu.SemaphoreType.DMA],
    )
    return f(qap, qfl, *krs, *vrs, dstg, srcg)


def _ssum_body(vals, idx8l, zeros_in, out, idx_v, val_v, tab):
    c = lax.axis_index("c")
    s = lax.axis_index("s")

    for ph in range(3):
        rows8, elo, ecnt = _PH8[ph]
        rps = rows8 // NS
        r0 = s * rps
        pltpu.sync_copy(zeros_in.at[pl.ds(0, rps)], tab.at[pl.ds(r0, rps)])
        plsc.subcore_barrier()

        per_s = ecnt // NS
        base0 = elo + s * per_s

        @pl.loop(0, per_s // CH)
        def _sc(i):
            base = base0 + i * CH
            pltpu.sync_copy(idx8l.at[pl.ds(base, CH)], idx_v)
            pltpu.sync_copy(vals.at[pl.ds(base, CH)], val_v)
            pltpu.sync_copy(val_v, tab.at[idx_v], add=True)

        plsc.subcore_barrier()

        half = rows8 // NC
        per_w = half // NS
        wbase = c * half + s * per_w
        pltpu.sync_copy(tab.at[pl.ds(wbase, per_w)],
                        out.at[pl.ds(_PH8BASE[ph] + wbase, per_w)])
        plsc.subcore_barrier()


def _ssum(vals128, idx8l, zeros_in):
    f = pl.kernel(
        _ssum_body,
        out_type=jax.ShapeDtypeStruct((R8TOT, 128), _f32),
        mesh=_MESH,
        scratch_types=[pltpu.VMEM((CH,), _i32),
                       pltpu.VMEM((CH, 128), _f32),
                       pltpu.VMEM_SHARED((_R8[1], 128), _f32)],
    )
    return f(vals128, idx8l, zeros_in)


def _sgather_body(tab, idx8, out, idx_v, row_v, sem):
    w = _wid()
    per_w = ET // NW
    base0 = w * per_w

    @pl.loop(0, per_w // CH)
    def _chunk(i):
        base = base0 + i * CH
        pltpu.sync_copy(idx8.at[pl.ds(base, CH)], idx_v)
        pltpu.async_copy(tab.at[idx_v], row_v, sem).wait()
        pltpu.sync_copy(row_v, out.at[pl.ds(base, CH)])


def _sgather(tab8, idx8):
    f = pl.kernel(
        _sgather_body,
        out_type=jax.ShapeDtypeStruct((ET, 128), _f32),
        mesh=_MESH,
        scratch_types=[pltpu.VMEM((CH,), _i32),
                       pltpu.VMEM((CH, 128), _f32),
                       pltpu.SemaphoreType.DMA],
    )
    return f(tab8, idx8)


_APE = 5 * EP
_AP_R = 10240
_FL_SP = 12544
_FL_R = 12800


def _smsg_body(m0, m1, idx_apd, idx_flg, zeros_in, oap, ofl,
               idx_v, val_v, tab):
    c = lax.axis_index("c")
    s = lax.axis_index("s")

    for chunk in range(2):
        m = m0 if chunk == 0 else m1

        rps = _AP_R // NS
        r0 = s * rps
        pltpu.sync_copy(zeros_in.at[pl.ds(0, rps)], tab.at[pl.ds(r0, rps)])
        plsc.subcore_barrier()
        half = _APE // NC
        per_s = half // NS
        base0 = c * half + s * per_s

        @pl.loop(0, per_s // CH)
        def _sa(i):
            b = base0 + i * CH
            mb = jnp.where(b < 3 * EP, b, b + EP)
            pltpu.sync_copy(idx_apd.at[pl.ds(b, CH)], idx_v)
            pltpu.sync_copy(m.at[pl.ds(mb, CH)], val_v)
            pltpu.sync_copy(val_v, tab.at[idx_v], add=True)

        plsc.subcore_barrier()
        pltpu.sync_copy(tab.at[pl.ds(r0, rps)],
                        oap.at[c, chunk, pl.ds(r0, rps)])
        plsc.subcore_barrier()

        for k in range(4):
            rps_f = _FL_R // NS
            rf0 = s * rps_f
            pltpu.sync_copy(zeros_in.at[pl.ds(0, rps_f)],
                            tab.at[pl.ds(rf0, rps_f)])
            plsc.subcore_barrier()
            halff = EP // NC
            per_f = halff // NS
            fbase0 = c * halff + s * per_f

            @pl.loop(0, per_f // CH)
            def _sf(i):
                b = fbase0 + i * CH
                pltpu.sync_copy(idx_flg.at[pl.ds(k * EP + b, CH)], idx_v)
                pltpu.sync_copy(m.at[pl.ds(3 * EP + b, CH)], val_v)
                pltpu.sync_copy(val_v, tab.at[idx_v], add=True)

            plsc.subcore_barrier()
            pltpu.sync_copy(tab.at[pl.ds(rf0, rps_f)],
                            ofl.at[c, chunk, pl.ds(k * _FL_R + rf0, rps_f)])
            plsc.subcore_barrier()


def _smsg(m0, m1, idx_apd, idx_flg, zeros_in):
    f = pl.kernel(
        _smsg_body,
        out_type=[jax.ShapeDtypeStruct((NC, 2, _AP_R, 128), _f32),
                  jax.ShapeDtypeStruct((NC, 2, 4 * _FL_R, 128), _f32)],
        mesh=_MESH,
        scratch_types=[pltpu.VMEM((CH,), _i32),
                       pltpu.VMEM((CH, 128), _f32),
                       pltpu.VMEM_SHARED((_FL_R, 128), _f32)],
    )
    return f(m0, m1, idx_apd, idx_flg, zeros_in)


def _blockdiag(rel, scale):
    z = jnp.zeros((HEADS, HEADS, DH, DH), _f32)
    idx = jnp.arange(HEADS)
    z = z.at[idx, idx].set(rel * scale[:, None, None])
    return z.transpose(0, 2, 1, 3).reshape(HID, HID)


def _pad1(a, n, val):
    return jnp.concatenate([a, jnp.full((n - a.shape[0],), val, a.dtype)])


def kernel(ap_x, fl_x, ap_h, params, edge0, edge1, edge2, edge3, edge4, edge5):
    p = params
    edges = [edge0, edge1, edge2, edge3, edge4, edge5]
    srcs = [e[0] for e in edges]
    dsts = [e[1] for e in edges]

    dstg = jnp.concatenate([_pad1(d, EP, 0) for d in dsts])
    srcg = jnp.concatenate([_pad1(s, EP, 0) for s in srcs])
    idxl = jnp.concatenate(
        [_pad1(d + _LOFF[ei], EP, _LDUM[ei]) for ei, d in enumerate(dsts)])
    idx8l = idxl // 8
    idx8g = idx8l + jnp.asarray(
        np.repeat(np.array(_G8BASE, np.int64), EP).astype(np.int32))
    idxl2 = idxl[:, None]
    zeros_in = jnp.zeros((_FL_R // NS, 128), _f32)
    idx_apd = jnp.concatenate(
        [_pad1(dsts[ei], EP, N_AP) for ei in (0, 1, 2, 4, 5)])
    fld = _pad1(dsts[3], EP, N_FL)
    idx_flg = jnp.concatenate([
        jnp.where((fld >= k * _FL_SP) & (fld < (k + 1) * _FL_SP),
                  fld - k * _FL_SP, _FL_SP) for k in range(4)])

    x_ap = _projln(ap_x, p["ap_proj_w"], p["ap_proj_b"],
                   p["ap_proj_g"], p["ap_proj_be"], 400)
    x_fl = _projln(fl_x, p["fl_proj_w"], p["fl_proj_b"],
                   p["fl_proj_g"], p["fl_proj_be"], 1000)

    inv_sqrt_dh = 1.0 / float(np.sqrt(DH))
    ones8 = jnp.ones((HEADS,), _f32)

    for l in range(LAYERS):
        big = {}
        for nt, xs in (("airport", x_ap), ("flight", x_fl)):
            eis = (0, 1, 2) if nt == "airport" else (3, 4, 5)
            Wk, bk = p[f"l{l}_k_{nt}_w"], p[f"l{l}_k_{nt}_b"]
            Wv, bv = p[f"l{l}_v_{nt}_w"], p[f"l{l}_v_{nt}_b"]
            Wq, bq = p[f"l{l}_q_{nt}_w"], p[f"l{l}_q_{nt}_b"]
            Ws, bs = [Wq], [bq]
            for ei in eis:
                bd = _blockdiag(p[f"l{l}_arel_{ei}"],
                                p[f"l{l}_prel_{ei}"] * inv_sqrt_dh)
                Ws.append(Wk @ bd)
                bs.append(bk @ bd)
            for ei in eis:
                bd = _blockdiag(p[f"l{l}_mrel_{ei}"], ones8)
                Ws.append(Wv @ bd)
                bs.append(bv @ bd)
            W = jnp.concatenate(Ws, axis=1)
            b = jnp.concatenate(bs)
            outs = _bigmat(xs, W, b, 400 if nt == "airport" else 1000)
            big[nt] = outs

        krs = list(big["airport"][1:4]) + list(big["flight"][1:4])
        vrs = list(big["airport"][4:7]) + list(big["flight"][4:7])
        qi, kj, vj = _g1(big["airport"][0], big["flight"][0],
                         krs, vrs, dstg, srcg)
        a16, a128 = _alpha16(qi, kj, idxl2, 1024)
        tab_a = _ssum(a128, idx8l, zeros_in)
        g_a = _sgather(tab_a, idx8g)
        e16, e128 = _e16(a16, g_a, idxl2, 1024)
        tab_e = _ssum(e128, idx8l, zeros_in)
        g_e = _sgather(tab_e, idx8g)
        m0, m1 = _msg(vj, e16, g_e, idxl2, 1024)
        agg_ap4, ofl = _smsg(m0, m1, idx_apd, idx_flg, zeros_in)
        agg_fl4 = jnp.concatenate(
            [ofl[:, :, k * _FL_R:k * _FL_R + _FL_SP] for k in range(4)],
            axis=2)

        for nt in ("airport", "flight"):
            a_sig = jnp.zeros((1, 128), _f32).at[0, 0].set(
                jax.nn.sigmoid(p[f"l{l}_skip_{nt}"]))
            if nt == "airport":
                x_ap = _node_update(agg_ap4, x_ap, p[f"l{l}_a_{nt}_w"],
                                    p[f"l{l}_a_{nt}_b"], a_sig,
                                    p[f"l{l}_ln_{nt}_g"], p[f"l{l}_ln_{nt}_b"], 400)
            else:
                x_fl = _node_update(agg_fl4, x_fl, p[f"l{l}_a_{nt}_w"],
                                    p[f"l{l}_a_{nt}_b"], a_sig,
                                    p[f"l{l}_ln_{nt}_g"], p[f"l{l}_ln_{nt}_b"], 1000)

    w2p = jnp.zeros((128, 128), _f32).at[:, 0].set(p["aph_w2"][:, 0])
    b2p = jnp.zeros((1, 128), _f32).at[0, 0].set(p["aph_b2"][0])
    ap_h_new, ap_p = _gru_head(x_ap, ap_h, p["gru_wih"], p["gru_whh"],
                               p["gru_bih"], p["gru_bhh"],
                               p["aph_w1"], p["aph_b1"], w2p, b2p, 400)
    ap_pred = ap_p[:, 0]

    flw2p = jnp.zeros((128, 128), _f32).at[:, 0].set(
        p["flh_w2"][:, 0] * 1.0)
    flw2p = flw2p.at[:, 1].set(0.0)
    cw1p = jnp.zeros((HID, 128), _f32).at[:, :64].set(p["flc_w1"])
    cb1p = jnp.zeros((1, 128), _f32).at[0, :64].set(p["flc_b1"])
    cw2p = jnp.zeros((128, 128), _f32).at[:64, 0].set(p["flc_w2"][:, 0])
    fl_p, fl_l = _fl_head(x_fl, p["gate_w"], p["gate_b"],
                          p["flh_w1"], p["flh_b1"], flw2p,
                          cw1p, cb1p, cw2p, 1000)
    fl_pred = fl_p[:, 0] + p["flh_b2"][0]
    fl_logits = fl_l[:, 0] + p["flc_b2"][0]
    return (ap_pred, fl_pred, fl_logits, ap_h_new)
